# baseline (speedup 1.0000x reference)
import jax
import jax.numpy as jnp
from jax import lax
from jax.experimental import pallas as pl
from jax.experimental.pallas import tpu as pltpu
from jax.experimental.pallas import tpu_sc as plsc

NUM_FACTORS = 32
BATCH = 16384
NC = 2
NS = 16
L = 16
NW = NC * NS
B_PER_W = BATCH // NW
CHUNK = 128
NCHUNK = B_PER_W // CHUNK
NGROUP = B_PER_W // L


def _gmf_body(uidx_hbm, iidx_hbm, utab_hbm, itab_hbm, w_hbm, b_hbm, out_hbm,
              uidx_v, iidx_v, urows_v, irows_v, w_v, b_v, t_v, out_v, sem):
    wid = lax.axis_index("s") * NC + lax.axis_index("c")
    base = wid * B_PER_W

    pltpu.sync_copy(uidx_hbm.at[wid], uidx_v)
    pltpu.sync_copy(iidx_hbm.at[wid], iidx_v)
    pltpu.sync_copy(w_hbm, w_v)
    pltpu.sync_copy(b_hbm, b_v)

    copies = []
    for j in range(NCHUNK):
        dst = pl.ds(j * CHUNK, CHUNK)
        copies.append(pltpu.async_copy(utab_hbm.at[uidx_v.at[j]],
                                       urows_v.at[dst], sem))
        copies.append(pltpu.async_copy(itab_hbm.at[iidx_v.at[j]],
                                       irows_v.at[dst], sem))
    for c in copies:
        c.wait()

    w0 = w_v[pl.ds(0, L)]
    w1 = w_v[pl.ds(L, L)]
    bias = b_v[...]
    lane = lax.iota(jnp.int32, L)

    def group(g, carry):
        rbase = g * L
        for b in range(L):
            r = rbase + b
            u0 = urows_v[r, pl.ds(0, L)]
            u1 = urows_v[r, pl.ds(L, L)]
            i0 = irows_v[r, pl.ds(0, L)]
            i1 = irows_v[r, pl.ds(L, L)]
            p = u0 * i0 * w0 + u1 * i1 * w1
            plsc.store_scatter(t_v, [lane * L + b], p)
        acc = t_v[pl.ds(0, L)]
        for k in range(1, L):
            acc = acc + t_v[pl.ds(k * L, L)]
        x = acc + bias
        out_v[pl.ds(rbase, L)] = 1.0 / (1.0 + jnp.exp(-x))
        return carry

    lax.fori_loop(0, NGROUP, group, 0)

    pltpu.sync_copy(out_v, out_hbm.at[pl.ds(base, B_PER_W)])


_gmf = pl.kernel(
    _gmf_body,
    out_type=jax.ShapeDtypeStruct((BATCH,), jnp.float32),
    mesh=plsc.VectorSubcoreMesh(core_axis_name="c", subcore_axis_name="s",
                                num_cores=NC, num_subcores=NS),
    compiler_params=pltpu.CompilerParams(needs_layout_passes=False,
                                         use_tc_tiling_on_sc=False),
    scratch_types=[
        pltpu.VMEM((NCHUNK, CHUNK), jnp.int32),
        pltpu.VMEM((NCHUNK, CHUNK), jnp.int32),
        pltpu.VMEM((B_PER_W, NUM_FACTORS), jnp.float32),
        pltpu.VMEM((B_PER_W, NUM_FACTORS), jnp.float32),
        pltpu.VMEM((NUM_FACTORS,), jnp.float32),
        pltpu.VMEM((L,), jnp.float32),
        pltpu.VMEM((L * L,), jnp.float32),
        pltpu.VMEM((B_PER_W,), jnp.float32),
        pltpu.SemaphoreType.DMA,
    ],
)


def kernel(user_indices, item_indices, user_table, item_table, affine_w, affine_b):
    uidx = user_indices.astype(jnp.int32).reshape(NW, NCHUNK, CHUNK)
    iidx = item_indices.astype(jnp.int32).reshape(NW, NCHUNK, CHUNK)
    w_flat = affine_w.reshape(NUM_FACTORS)
    b_b = jnp.broadcast_to(affine_b.reshape(1), (L,))
    return _gmf(uidx, iidx, user_table, item_table, w_flat, b_b)
